# Initial kernel scaffold; baseline (speedup 1.0000x reference)
import jax
import jax.numpy as jnp
from jax.experimental import pallas as pl


def kernel(x, table, W1, b1, W2, b2):
    raise NotImplementedError("write your pallas kernel here")


# speedup vs baseline: 5.6935x; 5.6935x over previous
import functools

import jax
import jax.numpy as jnp
from jax import lax
from jax.experimental import pallas as pl
from jax.experimental.pallas import tpu as pltpu
from jax.experimental.pallas import tpu_sc as plsc

VOCAB = 1000000
HIDDEN = 128
BATCH = 16384
HIST = 50

NC = 2
NS = 16
NW = NC * NS
LANES = 16
NCOL = HIDDEN // LANES

BAGS_PER_W = BATCH // NW
CHUNK = 2
ROWS = CHUNK * HIST
NCHUNK = BAGS_PER_W // CHUNK


def _bag_body(x_hbm, table_hbm, out_hbm, idx_v, rows_v, out_v, gsem0, gsem1):
    wid = lax.axis_index("s") * NC + lax.axis_index("c")
    bag_base = wid * BAGS_PER_W
    gsems = (gsem0, gsem1)

    pltpu.sync_copy(x_hbm.at[wid], idx_v)

    def issue(g, b):
        pltpu.async_copy(table_hbm.at[idx_v.at[g]], rows_v.at[b], gsems[b])

    def wait(g, b):
        pltpu.make_async_copy(
            table_hbm.at[idx_v.at[g]], rows_v.at[b], gsems[b]).wait()

    issue(0, 0)
    issue(1, 1)

    def outer(g0, carry):
        for b in range(2):
            g = g0 * 2 + b
            wait(g, b)

            @pl.when(g + 2 < NCHUNK)
            def _():
                issue(g + 2, b)

            buf = rows_v.at[b]
            for bag in range(CHUNK):
                base = bag * HIST

                def rbody(r, accs):
                    return tuple(
                        accs[c] + buf[base + r, pl.ds(c * LANES, LANES)]
                        for c in range(NCOL))

                init = tuple(
                    buf[base, pl.ds(c * LANES, LANES)] for c in range(NCOL))
                accs = lax.fori_loop(1, HIST, rbody, init)
                for c in range(NCOL):
                    out_v[g * CHUNK + bag, pl.ds(c * LANES, LANES)] = accs[c]
        return carry

    lax.fori_loop(0, NCHUNK // 2, outer, 0)

    pltpu.sync_copy(out_v, out_hbm.at[pl.ds(bag_base, BAGS_PER_W)])


_bag = functools.partial(
    pl.kernel,
    mesh=plsc.VectorSubcoreMesh(core_axis_name="c", subcore_axis_name="s"),
    out_type=jax.ShapeDtypeStruct((BATCH, HIDDEN), jnp.float32),
    scratch_types=[
        pltpu.VMEM((NCHUNK, ROWS), jnp.int32),
        pltpu.VMEM((2, ROWS, HIDDEN), jnp.float32),
        pltpu.VMEM((BAGS_PER_W, HIDDEN), jnp.float32),
        pltpu.SemaphoreType.DMA,
        pltpu.SemaphoreType.DMA,
    ],
)(_bag_body)


MLP_BLK = 2048


def _mlp_body(x_ref, w1_ref, b1_ref, w2_ref, b2_ref, o_ref):
    x = x_ref[...]
    dn = (((1,), (1,)), ((), ()))
    h = lax.dot_general(x, w1_ref[...], dn, preferred_element_type=jnp.float32)
    h = jnp.maximum(h + b1_ref[...], 0.0)
    o = lax.dot_general(h, w2_ref[...], dn, preferred_element_type=jnp.float32)
    o_ref[...] = o + b2_ref[...]


def _mlp(postemb, W1, b1, W2, b2):
    w_spec = pl.BlockSpec((HIDDEN, HIDDEN), lambda i: (0, 0))
    b_spec = pl.BlockSpec((1, HIDDEN), lambda i: (0, 0))
    return pl.pallas_call(
        _mlp_body,
        grid=(BATCH // MLP_BLK,),
        in_specs=[
            pl.BlockSpec((MLP_BLK, HIDDEN), lambda i: (i, 0)),
            w_spec, b_spec, w_spec, b_spec,
        ],
        out_specs=pl.BlockSpec((MLP_BLK, HIDDEN), lambda i: (i, 0)),
        out_shape=jax.ShapeDtypeStruct((BATCH, HIDDEN), jnp.float32),
    )(postemb, W1, b1.reshape(1, HIDDEN), W2, b2.reshape(1, HIDDEN))


def kernel(x, table, W1, b1, W2, b2):
    xr = x.astype(jnp.int32).reshape(NW, NCHUNK, ROWS)
    postemb = _bag(xr, table)
    return _mlp(postemb, W1, b1, W2, b2)
